# Initial kernel scaffold; baseline (speedup 1.0000x reference)
import jax
import jax.numpy as jnp
from jax.experimental import pallas as pl


def kernel(nfeat, edge_index, efeat, W, b, edge_emb):
    raise NotImplementedError("write your pallas kernel here")


# speedup vs baseline: 3.5793x; 3.5793x over previous
import functools

import jax
import jax.numpy as jnp
from jax import lax
from jax.experimental import pallas as pl
from jax.experimental.pallas import tpu as pltpu
from jax.experimental.pallas import tpu_sc as plsc

N_NODES = 10000
N_EDGES = 320000
DIM = 128

NUM_CORES = 2
NUM_TILES = 16
BLK = 128
SB = 8
BLOCKS_PER_TILE = 80
E_PAD = NUM_CORES * NUM_TILES * BLOCKS_PER_TILE * BLK
ACC_ROWS = 10112
ROWS_PER_TILE = ACC_ROWS // NUM_TILES
ZCHUNKS = (0, 128, 256, 384, 504)
CHUNKS = (128, 128, 128, 128, 120)
DEG_W = 16


def _sc_accumulate(nfeat, src2d, dst2d, ef0, ef1, ef2, e0, e1, e2, zf):
  mesh = plsc.VectorSubcoreMesh(core_axis_name="c", subcore_axis_name="s")

  @functools.partial(
      pl.kernel,
      out_type=[
          jax.ShapeDtypeStruct((NUM_CORES, ACC_ROWS, DIM), jnp.float32),
          jax.ShapeDtypeStruct((NUM_CORES, ACC_ROWS, DIM), jnp.float32),
          jax.ShapeDtypeStruct((NUM_CORES, 128, DIM), jnp.float32),
      ],
      mesh=mesh,
      scratch_types=[
          pltpu.VMEM_SHARED((ACC_ROWS, DIM), jnp.float32),
          pltpu.VMEM((SB, BLK), jnp.int32),
          pltpu.VMEM((SB, BLK), jnp.int32),
          pltpu.VMEM((SB, BLK), jnp.int32),
          pltpu.VMEM((SB, BLK), jnp.int32),
          pltpu.VMEM((SB, BLK), jnp.int32),
          pltpu.VMEM((BLK,), jnp.int32),
          pltpu.VMEM((BLK,), jnp.int32),
          pltpu.VMEM((BLK,), jnp.int32),
          pltpu.VMEM((BLK, DIM), jnp.float32),
          pltpu.VMEM((BLK, DIM), jnp.float32),
      ],
  )
  def kfn(nfeat_hbm, src_hbm, dst_hbm, ef0_hbm, ef1_hbm, ef2_hbm,
          e0_hbm, e1_hbm, e2_hbm, zf_hbm, pacc_hbm, pdeg_hbm,
          ct_hbm, acc_sh, src_v, dst_v, f0_v, f1_v, f2_v,
          code_v, src1d, dst1d, rows_a, ones_v):
    c = lax.axis_index("c")
    s = lax.axis_index("s")
    wid = c * NUM_TILES + s

    ones16 = jnp.ones((16,), jnp.float32)
    zeros16 = jnp.zeros((16,), jnp.float32)
    iota16 = lax.iota(jnp.int32, 16)

    base = s * ROWS_PER_TILE
    off = 0
    for n in CHUNKS:
      pltpu.sync_copy(zf_hbm.at[pl.ds(off, n)], rows_a.at[pl.ds(0, n)])
      pltpu.sync_copy(rows_a.at[pl.ds(0, n)], acc_sh.at[pl.ds(base + off, n)])
      off += n

    @pl.loop(0, BLK)
    def _(r):
      for j in range(DIM // 16):
        ones_v[r, pl.ds(j * 16, 16)] = ones16

    pltpu.sync_copy(e0_hbm, rows_a.at[pl.ds(16, 5)])
    pltpu.sync_copy(e1_hbm, rows_a.at[pl.ds(24, 5)])
    pltpu.sync_copy(e2_hbm, rows_a.at[pl.ds(32, 5)])

    @pl.loop(0, 8)
    def _(r):
      g = s * 8 + r
      i0 = jnp.minimum(g // 25, 4)
      i1 = (g // 5) % 5
      i2 = g % 5
      for jcol in range(DIM // 16):
        sl = pl.ds(jcol * 16, 16)
        rows_a[r, sl] = (rows_a[16 + i0, sl] + rows_a[24 + i1, sl]
                         + rows_a[32 + i2, sl])

    pltpu.sync_copy(rows_a.at[pl.ds(0, 8)], ct_hbm.at[c, pl.ds(s * 8, 8)])
    plsc.subcore_barrier()

    eb = wid * BLOCKS_PER_TILE

    @pl.loop(0, BLOCKS_PER_TILE // SB)
    def _(sb):
      sbsl = pl.ds(eb + sb * SB, SB)
      pltpu.sync_copy(src_hbm.at[sbsl], src_v)
      pltpu.sync_copy(dst_hbm.at[sbsl], dst_v)
      pltpu.sync_copy(ef0_hbm.at[sbsl], f0_v)
      pltpu.sync_copy(ef1_hbm.at[sbsl], f1_v)
      pltpu.sync_copy(ef2_hbm.at[sbsl], f2_v)

      @pl.loop(0, SB)
      def _(blk):
        for j in range(BLK // 16):
          sl = pl.ds(j * 16, 16)
          code_v[sl] = f0_v[blk, sl] * 25 + f1_v[blk, sl] * 5 + f2_v[blk, sl]
          src1d[sl] = src_v[blk, sl]
          dst1d[sl] = dst_v[blk, sl]
        pltpu.sync_copy(nfeat_hbm.at[src1d], rows_a)
        pltpu.sync_copy(rows_a, acc_sh.at[dst1d], add=True)
        pltpu.sync_copy(ct_hbm.at[c].at[code_v], rows_a)
        pltpu.sync_copy(rows_a, acc_sh.at[dst1d], add=True)

    plsc.subcore_barrier()

    off = 0
    for n in CHUNKS:
      sl = pl.ds(base + off, n)
      pltpu.sync_copy(acc_sh.at[sl], rows_a.at[pl.ds(0, n)])
      pltpu.sync_copy(rows_a.at[pl.ds(0, n)], pacc_hbm.at[c, sl])
      off += n

    off = 0
    for n in CHUNKS:
      pltpu.sync_copy(zf_hbm.at[pl.ds(off, n)], rows_a.at[pl.ds(0, n)])
      pltpu.sync_copy(rows_a.at[pl.ds(0, n)], acc_sh.at[pl.ds(base + off, n)])
      off += n
    plsc.subcore_barrier()

    @pl.loop(0, BLOCKS_PER_TILE // SB)
    def _(sb):
      sbsl = pl.ds(eb + sb * SB, SB)
      pltpu.sync_copy(dst_hbm.at[sbsl], dst_v)

      @pl.loop(0, SB)
      def _(blk):
        for j in range(BLK // 16):
          sl = pl.ds(j * 16, 16)
          dst1d[sl] = dst_v[blk, sl]
        pltpu.sync_copy(ones_v, acc_sh.at[dst1d], add=True)

    plsc.subcore_barrier()

    off = 0
    for n in CHUNKS:
      sl = pl.ds(base + off, n)
      pltpu.sync_copy(acc_sh.at[sl], rows_a.at[pl.ds(0, n)])
      pltpu.sync_copy(rows_a.at[pl.ds(0, n)], pdeg_hbm.at[c, sl])
      off += n

  return kfn(nfeat, src2d, dst2d, ef0, ef1, ef2, e0, e1, e2, zf)


def _tc_finish_body(nfeat_ref, pacc_ref, pdeg0_ref, pdeg1_ref, w_ref, b_ref,
                    out_ref):
  neigh = pacc_ref[0, 0:N_NODES, :] + pacc_ref[1, 0:N_NODES, :]
  deg = pdeg0_ref[0:N_NODES, 0] + pdeg1_ref[0:N_NODES, 0] + 1.0
  h = (nfeat_ref[...] + neigh) / deg[:, None]
  out_ref[...] = (
      lax.dot_general(h, w_ref[...], (((1,), (1,)), ((), ())),
                      precision=lax.Precision.HIGHEST,
                      preferred_element_type=jnp.float32)
      + b_ref[...][None, :]
  )


@jax.jit
def kernel(nfeat, edge_index, efeat, W, b, edge_emb):
  pad = E_PAD - N_EDGES
  src = jnp.pad(edge_index[0].astype(jnp.int32), (0, pad)).reshape(-1, BLK)
  dst = jnp.pad(edge_index[1].astype(jnp.int32), (0, pad),
                constant_values=N_NODES).reshape(-1, BLK)
  ef = efeat.astype(jnp.int32)
  ef0 = jnp.pad(ef[:, 0], (0, pad)).reshape(-1, BLK)
  ef1 = jnp.pad(ef[:, 1], (0, pad)).reshape(-1, BLK)
  ef2 = jnp.pad(ef[:, 2], (0, pad)).reshape(-1, BLK)
  e0 = edge_emb[0]
  e1 = edge_emb[1]
  e2 = edge_emb[2]
  zf = jnp.zeros((ROWS_PER_TILE, DIM), jnp.float32)

  pacc, pdeg, _ct = _sc_accumulate(nfeat, src, dst, ef0, ef1, ef2, e0, e1, e2,
                                   zf)

  out = pl.pallas_call(
      _tc_finish_body,
      out_shape=jax.ShapeDtypeStruct((N_NODES, DIM), jnp.float32),
  )(nfeat, pacc, pdeg[0], pdeg[1], W, b)
  return out


# speedup vs baseline: 5.1442x; 1.4372x over previous
import functools

import jax
import jax.numpy as jnp
from jax import lax
from jax.experimental import pallas as pl
from jax.experimental.pallas import tpu as pltpu
from jax.experimental.pallas import tpu_sc as plsc

N_NODES = 10000
N_EDGES = 320000
DIM = 128

NUM_CORES = 2
NUM_TILES = 16
BLK = 128
SB = 8
BLOCKS_PER_TILE = 80
E_PAD = NUM_CORES * NUM_TILES * BLOCKS_PER_TILE * BLK
ACC_ROWS = 10112
ROWS_PER_TILE = ACC_ROWS // NUM_TILES
ZCHUNKS = (0, 128, 256, 384, 504)
CHUNKS = (128, 128, 128, 128, 120)
DEG_W = 16


def _sc_accumulate(nfeat, src2d, dst2d, ef0, ef1, ef2, e0, e1, e2, zf):
  mesh = plsc.VectorSubcoreMesh(core_axis_name="c", subcore_axis_name="s")

  @functools.partial(
      pl.kernel,
      out_type=[
          jax.ShapeDtypeStruct((NUM_CORES, ACC_ROWS, DIM), jnp.float32),
          jax.ShapeDtypeStruct((NUM_CORES, ACC_ROWS, DIM), jnp.float32),
          jax.ShapeDtypeStruct((NUM_CORES, 128, DIM), jnp.float32),
      ],
      mesh=mesh,
      scratch_types=[
          pltpu.VMEM_SHARED((ACC_ROWS, DIM), jnp.float32),
          pltpu.VMEM((SB, BLK), jnp.int32),
          pltpu.VMEM((SB, BLK), jnp.int32),
          pltpu.VMEM((SB, BLK), jnp.int32),
          pltpu.VMEM((SB, BLK), jnp.int32),
          pltpu.VMEM((SB, BLK), jnp.int32),
          pltpu.VMEM((BLK,), jnp.int32),
          pltpu.VMEM((BLK,), jnp.int32),
          pltpu.VMEM((BLK,), jnp.int32),
          pltpu.VMEM((BLK, DIM), jnp.float32),
          pltpu.VMEM((BLK, DIM), jnp.float32),
          pltpu.SemaphoreType.DMA,
          pltpu.SemaphoreType.DMA,
      ],
  )
  def kfn(nfeat_hbm, src_hbm, dst_hbm, ef0_hbm, ef1_hbm, ef2_hbm,
          e0_hbm, e1_hbm, e2_hbm, zf_hbm, pacc_hbm, pdeg_hbm,
          ct_hbm, acc_sh, src_v, dst_v, f0_v, f1_v, f2_v,
          code_v, src1d, dst1d, rows_a, ones_v, sem_a, sem_b):
    c = lax.axis_index("c")
    s = lax.axis_index("s")
    wid = c * NUM_TILES + s

    ones16 = jnp.ones((16,), jnp.float32)
    zeros16 = jnp.zeros((16,), jnp.float32)
    iota16 = lax.iota(jnp.int32, 16)

    base = s * ROWS_PER_TILE
    off = 0
    for n in CHUNKS:
      pltpu.sync_copy(zf_hbm.at[pl.ds(off, n)], rows_a.at[pl.ds(0, n)])
      pltpu.sync_copy(rows_a.at[pl.ds(0, n)], acc_sh.at[pl.ds(base + off, n)])
      off += n

    pltpu.sync_copy(e0_hbm, rows_a.at[pl.ds(16, 5)])
    pltpu.sync_copy(e1_hbm, rows_a.at[pl.ds(24, 5)])
    pltpu.sync_copy(e2_hbm, rows_a.at[pl.ds(32, 5)])

    @pl.loop(0, 8)
    def _(r):
      g = s * 8 + r
      i0 = jnp.minimum(g // 25, 4)
      i1 = (g // 5) % 5
      i2 = g % 5
      for jcol in range(DIM // 16):
        sl = pl.ds(jcol * 16, 16)
        rows_a[r, sl] = (rows_a[16 + i0, sl] + rows_a[24 + i1, sl]
                         + rows_a[32 + i2, sl])

    pltpu.sync_copy(rows_a.at[pl.ds(0, 8)], ct_hbm.at[c, pl.ds(s * 8, 8)])
    plsc.subcore_barrier()

    eb = wid * BLOCKS_PER_TILE

    @pl.loop(0, BLOCKS_PER_TILE // SB)
    def _(sb):
      sbsl = pl.ds(eb + sb * SB, SB)
      pltpu.sync_copy(src_hbm.at[sbsl], src_v)
      pltpu.sync_copy(dst_hbm.at[sbsl], dst_v)
      pltpu.sync_copy(ef0_hbm.at[sbsl], f0_v)
      pltpu.sync_copy(ef1_hbm.at[sbsl], f1_v)
      pltpu.sync_copy(ef2_hbm.at[sbsl], f2_v)

      @pl.loop(0, SB)
      def _(blk):
        for j in range(BLK // 16):
          sl = pl.ds(j * 16, 16)
          code_v[sl] = f0_v[blk, sl] * 25 + f1_v[blk, sl] * 5 + f2_v[blk, sl]
          src1d[sl] = src_v[blk, sl]
          dst1d[sl] = dst_v[blk, sl]
        cp_a = pltpu.async_copy(nfeat_hbm.at[src1d], rows_a, sem_a)
        cp_b = pltpu.async_copy(ct_hbm.at[c].at[code_v], ones_v, sem_b)
        cp_a.wait()
        cp_b.wait()
        sp_a = pltpu.async_copy(rows_a, acc_sh.at[dst1d], sem_a, add=True)
        sp_b = pltpu.async_copy(ones_v, acc_sh.at[dst1d], sem_b, add=True)
        sp_a.wait()
        sp_b.wait()

    plsc.subcore_barrier()

    off = 0
    for n in CHUNKS:
      sl = pl.ds(base + off, n)
      pltpu.sync_copy(acc_sh.at[sl], rows_a.at[pl.ds(0, n)])
      pltpu.sync_copy(rows_a.at[pl.ds(0, n)], pacc_hbm.at[c, sl])
      off += n

    @pl.loop(0, BLK)
    def _(r):
      for j in range(DIM // 16):
        ones_v[r, pl.ds(j * 16, 16)] = ones16

    off = 0
    for n in CHUNKS:
      pltpu.sync_copy(zf_hbm.at[pl.ds(off, n)], rows_a.at[pl.ds(0, n)])
      pltpu.sync_copy(rows_a.at[pl.ds(0, n)], acc_sh.at[pl.ds(base + off, n)])
      off += n
    plsc.subcore_barrier()

    @pl.loop(0, BLOCKS_PER_TILE // SB)
    def _(sb):
      sbsl = pl.ds(eb + sb * SB, SB)
      pltpu.sync_copy(dst_hbm.at[sbsl], dst_v)

      cps = [None, None]
      bufs = (dst1d, src1d)
      sems = (sem_a, sem_b)
      for b in range(SB):
        p = b % 2
        if cps[p] is not None:
          cps[p].wait()
        for j in range(BLK // 16):
          sl = pl.ds(j * 16, 16)
          bufs[p][sl] = dst_v[b, sl]
        cps[p] = pltpu.async_copy(ones_v, acc_sh.at[bufs[p]], sems[p],
                                  add=True)
      cps[0].wait()
      cps[1].wait()

    plsc.subcore_barrier()

    off = 0
    for n in CHUNKS:
      sl = pl.ds(base + off, n)
      pltpu.sync_copy(acc_sh.at[sl], rows_a.at[pl.ds(0, n)])
      pltpu.sync_copy(rows_a.at[pl.ds(0, n)], pdeg_hbm.at[c, sl])
      off += n

  return kfn(nfeat, src2d, dst2d, ef0, ef1, ef2, e0, e1, e2, zf)


def _tc_finish_body(nfeat_ref, pacc_ref, pdeg0_ref, pdeg1_ref, w_ref, b_ref,
                    out_ref):
  neigh = pacc_ref[0, 0:N_NODES, :] + pacc_ref[1, 0:N_NODES, :]
  deg = pdeg0_ref[0:N_NODES, 0] + pdeg1_ref[0:N_NODES, 0] + 1.0
  h = (nfeat_ref[...] + neigh) / deg[:, None]
  out_ref[...] = (
      lax.dot_general(h, w_ref[...], (((1,), (1,)), ((), ())),
                      precision=lax.Precision.HIGHEST,
                      preferred_element_type=jnp.float32)
      + b_ref[...][None, :]
  )


@jax.jit
def kernel(nfeat, edge_index, efeat, W, b, edge_emb):
  pad = E_PAD - N_EDGES
  src = jnp.pad(edge_index[0].astype(jnp.int32), (0, pad)).reshape(-1, BLK)
  dst = jnp.pad(edge_index[1].astype(jnp.int32), (0, pad),
                constant_values=N_NODES).reshape(-1, BLK)
  ef = efeat.astype(jnp.int32)
  ef0 = jnp.pad(ef[:, 0], (0, pad)).reshape(-1, BLK)
  ef1 = jnp.pad(ef[:, 1], (0, pad)).reshape(-1, BLK)
  ef2 = jnp.pad(ef[:, 2], (0, pad)).reshape(-1, BLK)
  e0 = edge_emb[0]
  e1 = edge_emb[1]
  e2 = edge_emb[2]
  zf = jnp.zeros((ROWS_PER_TILE, DIM), jnp.float32)

  pacc, pdeg, _ct = _sc_accumulate(nfeat, src, dst, ef0, ef1, ef2, e0, e1, e2,
                                   zf)

  out = pl.pallas_call(
      _tc_finish_body,
      out_shape=jax.ShapeDtypeStruct((N_NODES, DIM), jnp.float32),
  )(nfeat, pacc, pdeg[0], pdeg[1], W, b)
  return out


# speedup vs baseline: 5.6530x; 1.0989x over previous
import functools

import jax
import jax.numpy as jnp
from jax import lax
from jax.experimental import pallas as pl
from jax.experimental.pallas import tpu as pltpu
from jax.experimental.pallas import tpu_sc as plsc

N_NODES = 10000
N_EDGES = 320000
DIM = 128

NUM_CORES = 2
NUM_TILES = 16
BLK = 128
SB = 8
BLOCKS_PER_TILE = 80
K0 = 96
K1 = 64
E_PAD = NUM_TILES * (K0 + K1) * BLK
ACC_ROWS = 10112
ROWS_PER_TILE = ACC_ROWS // NUM_TILES
ZCHUNKS = (0, 128, 256, 384, 504)
CHUNKS = (128, 128, 128, 128, 120)
DEG_W = 16


def _sc_accumulate(nfeat, src2d, dst2d, ef0, ef1, ef2, e0, e1, e2, zf):
  mesh = plsc.VectorSubcoreMesh(core_axis_name="c", subcore_axis_name="s")

  @functools.partial(
      pl.kernel,
      out_type=[
          jax.ShapeDtypeStruct((NUM_CORES, ACC_ROWS, DIM), jnp.float32),
          jax.ShapeDtypeStruct((NUM_CORES, ACC_ROWS, DIM), jnp.float32),
          jax.ShapeDtypeStruct((NUM_CORES, 128, DIM), jnp.float32),
      ],
      mesh=mesh,
      scratch_types=[
          pltpu.VMEM_SHARED((ACC_ROWS, DIM), jnp.float32),
          pltpu.VMEM((SB, BLK), jnp.int32),
          pltpu.VMEM((SB, BLK), jnp.int32),
          pltpu.VMEM((SB, BLK), jnp.int32),
          pltpu.VMEM((SB, BLK), jnp.int32),
          pltpu.VMEM((SB, BLK), jnp.int32),
          pltpu.VMEM((BLK,), jnp.int32),
          pltpu.VMEM((BLK,), jnp.int32),
          pltpu.VMEM((BLK,), jnp.int32),
          pltpu.VMEM((BLK, DIM), jnp.float32),
          pltpu.VMEM((BLK, DIM), jnp.float32),
          pltpu.SemaphoreType.DMA,
          pltpu.SemaphoreType.DMA,
      ],
  )
  def kfn(nfeat_hbm, src_hbm, dst_hbm, ef0_hbm, ef1_hbm, ef2_hbm,
          e0_hbm, e1_hbm, e2_hbm, zf_hbm, pacc_hbm, pdeg_hbm,
          ct_hbm, acc_sh, src_v, dst_v, f0_v, f1_v, f2_v,
          code_v, src1d, dst1d, rows_a, ones_v, sem_a, sem_b):
    c = lax.axis_index("c")
    s = lax.axis_index("s")
    wid = c * NUM_TILES + s

    ones16 = jnp.ones((16,), jnp.float32)
    zeros16 = jnp.zeros((16,), jnp.float32)
    iota16 = lax.iota(jnp.int32, 16)

    base = s * ROWS_PER_TILE
    off = 0
    for n in CHUNKS:
      pltpu.sync_copy(zf_hbm.at[pl.ds(off, n)], rows_a.at[pl.ds(0, n)])
      pltpu.sync_copy(rows_a.at[pl.ds(0, n)], acc_sh.at[pl.ds(base + off, n)])
      off += n

    pltpu.sync_copy(e0_hbm, rows_a.at[pl.ds(16, 5)])
    pltpu.sync_copy(e1_hbm, rows_a.at[pl.ds(24, 5)])
    pltpu.sync_copy(e2_hbm, rows_a.at[pl.ds(32, 5)])

    @pl.loop(0, 8)
    def _(r):
      g = s * 8 + r
      i0 = jnp.minimum(g // 25, 4)
      i1 = (g // 5) % 5
      i2 = g % 5
      for jcol in range(DIM // 16):
        sl = pl.ds(jcol * 16, 16)
        rows_a[r, sl] = (rows_a[16 + i0, sl] + rows_a[24 + i1, sl]
                         + rows_a[32 + i2, sl])

    pltpu.sync_copy(rows_a.at[pl.ds(0, 8)], ct_hbm.at[c, pl.ds(s * 8, 8)])
    plsc.subcore_barrier()

    nblocks = K0 + c * (K1 - K0)
    eb = c * NUM_TILES * K0 + s * nblocks

    @pl.loop(0, nblocks // SB)
    def _(sb):
      sbsl = pl.ds(eb + sb * SB, SB)
      pltpu.sync_copy(src_hbm.at[sbsl], src_v)
      pltpu.sync_copy(dst_hbm.at[sbsl], dst_v)
      pltpu.sync_copy(ef0_hbm.at[sbsl], f0_v)
      pltpu.sync_copy(ef1_hbm.at[sbsl], f1_v)
      pltpu.sync_copy(ef2_hbm.at[sbsl], f2_v)

      @pl.loop(0, SB)
      def _(blk):
        for j in range(BLK // 16):
          sl = pl.ds(j * 16, 16)
          code_v[sl] = f0_v[blk, sl] * 25 + f1_v[blk, sl] * 5 + f2_v[blk, sl]
          src1d[sl] = src_v[blk, sl]
          dst1d[sl] = dst_v[blk, sl]
        cp_a = pltpu.async_copy(nfeat_hbm.at[src1d], rows_a, sem_a)
        cp_b = pltpu.async_copy(ct_hbm.at[c].at[code_v], ones_v, sem_b)
        cp_a.wait()
        cp_b.wait()
        sp_a = pltpu.async_copy(rows_a, acc_sh.at[dst1d], sem_a, add=True)
        sp_b = pltpu.async_copy(ones_v, acc_sh.at[dst1d], sem_b, add=True)
        sp_a.wait()
        sp_b.wait()

    plsc.subcore_barrier()

    off = 0
    for n in CHUNKS:
      sl = pl.ds(base + off, n)
      pltpu.sync_copy(acc_sh.at[sl], rows_a.at[pl.ds(0, n)])
      pltpu.sync_copy(rows_a.at[pl.ds(0, n)], pacc_hbm.at[c, sl])
      off += n

    @pl.loop(0, BLK)
    def _(r):
      for j in range(DIM // 16):
        ones_v[r, pl.ds(j * 16, 16)] = ones16

    off = 0
    for n in CHUNKS:
      pltpu.sync_copy(zf_hbm.at[pl.ds(off, n)], rows_a.at[pl.ds(0, n)])
      pltpu.sync_copy(rows_a.at[pl.ds(0, n)], acc_sh.at[pl.ds(base + off, n)])
      off += n
    plsc.subcore_barrier()

    @pl.loop(0, nblocks // SB)
    def _(sb):
      sbsl = pl.ds(eb + sb * SB, SB)
      pltpu.sync_copy(dst_hbm.at[sbsl], dst_v)

      cps = [None, None]
      bufs = (dst1d, src1d)
      sems = (sem_a, sem_b)
      for b in range(SB):
        p = b % 2
        if cps[p] is not None:
          cps[p].wait()
        for j in range(BLK // 16):
          sl = pl.ds(j * 16, 16)
          bufs[p][sl] = dst_v[b, sl]
        cps[p] = pltpu.async_copy(ones_v, acc_sh.at[bufs[p]], sems[p],
                                  add=True)
      cps[0].wait()
      cps[1].wait()

    plsc.subcore_barrier()

    off = 0
    for n in CHUNKS:
      sl = pl.ds(base + off, n)
      pltpu.sync_copy(acc_sh.at[sl], rows_a.at[pl.ds(0, n)])
      pltpu.sync_copy(rows_a.at[pl.ds(0, n)], pdeg_hbm.at[c, sl])
      off += n

  return kfn(nfeat, src2d, dst2d, ef0, ef1, ef2, e0, e1, e2, zf)


def _tc_finish_body(nfeat_ref, pacc_ref, pdeg0_ref, pdeg1_ref, w_ref, b_ref,
                    out_ref):
  neigh = pacc_ref[0, 0:N_NODES, :] + pacc_ref[1, 0:N_NODES, :]
  deg = pdeg0_ref[0:N_NODES, 0] + pdeg1_ref[0:N_NODES, 0] + 1.0
  h = (nfeat_ref[...] + neigh) / deg[:, None]
  out_ref[...] = (
      lax.dot_general(h, w_ref[...], (((1,), (1,)), ((), ())),
                      precision=lax.Precision.HIGHEST,
                      preferred_element_type=jnp.float32)
      + b_ref[...][None, :]
  )


@jax.jit
def kernel(nfeat, edge_index, efeat, W, b, edge_emb):
  pad = E_PAD - N_EDGES
  src = jnp.pad(edge_index[0].astype(jnp.int32), (0, pad)).reshape(-1, BLK)
  dst = jnp.pad(edge_index[1].astype(jnp.int32), (0, pad),
                constant_values=N_NODES).reshape(-1, BLK)
  ef = efeat.astype(jnp.int32)
  ef0 = jnp.pad(ef[:, 0], (0, pad)).reshape(-1, BLK)
  ef1 = jnp.pad(ef[:, 1], (0, pad)).reshape(-1, BLK)
  ef2 = jnp.pad(ef[:, 2], (0, pad)).reshape(-1, BLK)
  e0 = edge_emb[0]
  e1 = edge_emb[1]
  e2 = edge_emb[2]
  zf = jnp.zeros((ROWS_PER_TILE, DIM), jnp.float32)

  pacc, pdeg, _ct = _sc_accumulate(nfeat, src, dst, ef0, ef1, ef2, e0, e1, e2,
                                   zf)

  out = pl.pallas_call(
      _tc_finish_body,
      out_shape=jax.ShapeDtypeStruct((N_NODES, DIM), jnp.float32),
  )(nfeat, pacc, pdeg[0], pdeg[1], W, b)
  return out


# speedup vs baseline: 5.9832x; 1.0584x over previous
import functools

import jax
import jax.numpy as jnp
from jax import lax
from jax.experimental import pallas as pl
from jax.experimental.pallas import tpu as pltpu
from jax.experimental.pallas import tpu_sc as plsc

N_NODES = 10000
N_EDGES = 320000
DIM = 128

NUM_CORES = 2
NUM_TILES = 16
BLK = 128
SB = 8
BLOCKS_PER_TILE = 80
K0 = 104
K1 = 56
E_PAD = NUM_TILES * (K0 + K1) * BLK
ACC_ROWS = 10112
ROWS_PER_TILE = ACC_ROWS // NUM_TILES
ZCHUNKS = (0, 128, 256, 384, 504)
CHUNKS = (128, 128, 128, 128, 120)
DEG_W = 16


def _sc_accumulate(nfeat, src2d, dst2d, ef0, ef1, ef2, e0, e1, e2, zf):
  mesh = plsc.VectorSubcoreMesh(core_axis_name="c", subcore_axis_name="s")

  @functools.partial(
      pl.kernel,
      out_type=[
          jax.ShapeDtypeStruct((NUM_CORES, ACC_ROWS, DIM), jnp.float32),
          jax.ShapeDtypeStruct((NUM_CORES, ACC_ROWS, DIM), jnp.float32),
          jax.ShapeDtypeStruct((NUM_CORES, 128, DIM), jnp.float32),
      ],
      mesh=mesh,
      scratch_types=[
          pltpu.VMEM_SHARED((ACC_ROWS, DIM), jnp.float32),
          pltpu.VMEM((SB, BLK), jnp.int32),
          pltpu.VMEM((SB, BLK), jnp.int32),
          pltpu.VMEM((SB, BLK), jnp.int32),
          pltpu.VMEM((SB, BLK), jnp.int32),
          pltpu.VMEM((SB, BLK), jnp.int32),
          pltpu.VMEM((BLK,), jnp.int32),
          pltpu.VMEM((BLK,), jnp.int32),
          pltpu.VMEM((BLK,), jnp.int32),
          pltpu.VMEM((BLK, DIM), jnp.float32),
          pltpu.VMEM((BLK, DIM), jnp.float32),
          pltpu.SemaphoreType.DMA,
          pltpu.SemaphoreType.DMA,
      ],
  )
  def kfn(nfeat_hbm, src_hbm, dst_hbm, ef0_hbm, ef1_hbm, ef2_hbm,
          e0_hbm, e1_hbm, e2_hbm, zf_hbm, pacc_hbm, pdeg_hbm,
          ct_hbm, acc_sh, src_v, dst_v, f0_v, f1_v, f2_v,
          code_v, src1d, dst1d, rows_a, ones_v, sem_a, sem_b):
    c = lax.axis_index("c")
    s = lax.axis_index("s")
    wid = c * NUM_TILES + s

    ones16 = jnp.ones((16,), jnp.float32)
    zeros16 = jnp.zeros((16,), jnp.float32)
    iota16 = lax.iota(jnp.int32, 16)

    base = s * ROWS_PER_TILE
    off = 0
    for n in CHUNKS:
      pltpu.sync_copy(zf_hbm.at[pl.ds(off, n)], rows_a.at[pl.ds(0, n)])
      pltpu.sync_copy(rows_a.at[pl.ds(0, n)], acc_sh.at[pl.ds(base + off, n)])
      off += n

    pltpu.sync_copy(e0_hbm, rows_a.at[pl.ds(16, 5)])
    pltpu.sync_copy(e1_hbm, rows_a.at[pl.ds(24, 5)])
    pltpu.sync_copy(e2_hbm, rows_a.at[pl.ds(32, 5)])

    @pl.loop(0, 8)
    def _(r):
      g = s * 8 + r
      i0 = jnp.minimum(g // 25, 4)
      i1 = (g // 5) % 5
      i2 = g % 5
      for jcol in range(DIM // 16):
        sl = pl.ds(jcol * 16, 16)
        rows_a[r, sl] = (rows_a[16 + i0, sl] + rows_a[24 + i1, sl]
                         + rows_a[32 + i2, sl])

    pltpu.sync_copy(rows_a.at[pl.ds(0, 8)], ct_hbm.at[c, pl.ds(s * 8, 8)])
    plsc.subcore_barrier()

    nblocks = K0 + c * (K1 - K0)
    eb = c * NUM_TILES * K0 + s * nblocks

    @pl.loop(0, nblocks // SB)
    def _(sb):
      sbsl = pl.ds(eb + sb * SB, SB)
      pltpu.sync_copy(src_hbm.at[sbsl], src_v)
      pltpu.sync_copy(dst_hbm.at[sbsl], dst_v)
      pltpu.sync_copy(ef0_hbm.at[sbsl], f0_v)
      pltpu.sync_copy(ef1_hbm.at[sbsl], f1_v)
      pltpu.sync_copy(ef2_hbm.at[sbsl], f2_v)

      @pl.loop(0, SB)
      def _(blk):
        for j in range(BLK // 16):
          sl = pl.ds(j * 16, 16)
          code_v[sl] = f0_v[blk, sl] * 25 + f1_v[blk, sl] * 5 + f2_v[blk, sl]
          src1d[sl] = src_v[blk, sl]
          dst1d[sl] = dst_v[blk, sl]
        cp_a = pltpu.async_copy(nfeat_hbm.at[src1d], rows_a, sem_a)
        cp_b = pltpu.async_copy(ct_hbm.at[c].at[code_v], ones_v, sem_b)
        cp_a.wait()
        cp_b.wait()
        sp_a = pltpu.async_copy(rows_a, acc_sh.at[dst1d], sem_a, add=True)
        sp_b = pltpu.async_copy(ones_v, acc_sh.at[dst1d], sem_b, add=True)
        sp_a.wait()
        sp_b.wait()

    plsc.subcore_barrier()

    off = 0
    for n in CHUNKS:
      sl = pl.ds(base + off, n)
      pltpu.sync_copy(acc_sh.at[sl], rows_a.at[pl.ds(0, n)])
      pltpu.sync_copy(rows_a.at[pl.ds(0, n)], pacc_hbm.at[c, sl])
      off += n

    @pl.loop(0, BLK)
    def _(r):
      for j in range(DIM // 16):
        ones_v[r, pl.ds(j * 16, 16)] = ones16

    off = 0
    for n in CHUNKS:
      pltpu.sync_copy(zf_hbm.at[pl.ds(off, n)], rows_a.at[pl.ds(0, n)])
      pltpu.sync_copy(rows_a.at[pl.ds(0, n)], acc_sh.at[pl.ds(base + off, n)])
      off += n
    plsc.subcore_barrier()

    @pl.loop(0, nblocks // SB)
    def _(sb):
      sbsl = pl.ds(eb + sb * SB, SB)
      pltpu.sync_copy(dst_hbm.at[sbsl], dst_v)

      cps = [None, None]
      bufs = (dst1d, src1d)
      sems = (sem_a, sem_b)
      for b in range(SB):
        p = b % 2
        if cps[p] is not None:
          cps[p].wait()
        for j in range(BLK // 16):
          sl = pl.ds(j * 16, 16)
          bufs[p][sl] = dst_v[b, sl]
        cps[p] = pltpu.async_copy(ones_v, acc_sh.at[bufs[p]], sems[p],
                                  add=True)
      cps[0].wait()
      cps[1].wait()

    plsc.subcore_barrier()

    off = 0
    for n in CHUNKS:
      sl = pl.ds(base + off, n)
      pltpu.sync_copy(acc_sh.at[sl], rows_a.at[pl.ds(0, n)])
      pltpu.sync_copy(rows_a.at[pl.ds(0, n)], pdeg_hbm.at[c, sl])
      off += n

  return kfn(nfeat, src2d, dst2d, ef0, ef1, ef2, e0, e1, e2, zf)


def _tc_finish_body(nfeat_ref, pacc_ref, pdeg0_ref, pdeg1_ref, w_ref, b_ref,
                    out_ref):
  neigh = pacc_ref[0, 0:N_NODES, :] + pacc_ref[1, 0:N_NODES, :]
  deg = pdeg0_ref[0:N_NODES, 0] + pdeg1_ref[0:N_NODES, 0] + 1.0
  h = (nfeat_ref[...] + neigh) / deg[:, None]
  out_ref[...] = (
      lax.dot_general(h, w_ref[...], (((1,), (1,)), ((), ())),
                      precision=lax.Precision.HIGHEST,
                      preferred_element_type=jnp.float32)
      + b_ref[...][None, :]
  )


@jax.jit
def kernel(nfeat, edge_index, efeat, W, b, edge_emb):
  pad = E_PAD - N_EDGES
  src = jnp.pad(edge_index[0].astype(jnp.int32), (0, pad)).reshape(-1, BLK)
  dst = jnp.pad(edge_index[1].astype(jnp.int32), (0, pad),
                constant_values=N_NODES).reshape(-1, BLK)
  ef = efeat.astype(jnp.int32)
  ef0 = jnp.pad(ef[:, 0], (0, pad)).reshape(-1, BLK)
  ef1 = jnp.pad(ef[:, 1], (0, pad)).reshape(-1, BLK)
  ef2 = jnp.pad(ef[:, 2], (0, pad)).reshape(-1, BLK)
  e0 = edge_emb[0]
  e1 = edge_emb[1]
  e2 = edge_emb[2]
  zf = jnp.zeros((ROWS_PER_TILE, DIM), jnp.float32)

  pacc, pdeg, _ct = _sc_accumulate(nfeat, src, dst, ef0, ef1, ef2, e0, e1, e2,
                                   zf)

  out = pl.pallas_call(
      _tc_finish_body,
      out_shape=jax.ShapeDtypeStruct((N_NODES, DIM), jnp.float32),
  )(nfeat, pacc, pdeg[0], pdeg[1], W, b)
  return out


# speedup vs baseline: 6.2655x; 1.0472x over previous
import functools

import jax
import jax.numpy as jnp
from jax import lax
from jax.experimental import pallas as pl
from jax.experimental.pallas import tpu as pltpu
from jax.experimental.pallas import tpu_sc as plsc

N_NODES = 10000
N_EDGES = 320000
DIM = 128

NUM_CORES = 2
NUM_TILES = 16
BLK = 128
SB = 8
BLOCKS_PER_TILE = 80
K0 = 112
K1 = 48
E_PAD = NUM_TILES * (K0 + K1) * BLK
ACC_ROWS = 10112
ROWS_PER_TILE = ACC_ROWS // NUM_TILES
ZCHUNKS = (0, 128, 256, 384, 504)
CHUNKS = (128, 128, 128, 128, 120)
DEG_W = 16


def _sc_accumulate(nfeat, src2d, dst2d, ef0, ef1, ef2, e0, e1, e2, zf):
  mesh = plsc.VectorSubcoreMesh(core_axis_name="c", subcore_axis_name="s")

  @functools.partial(
      pl.kernel,
      out_type=[
          jax.ShapeDtypeStruct((NUM_CORES, ACC_ROWS, DIM), jnp.float32),
          jax.ShapeDtypeStruct((NUM_CORES, ACC_ROWS, DIM), jnp.float32),
          jax.ShapeDtypeStruct((NUM_CORES, 128, DIM), jnp.float32),
      ],
      mesh=mesh,
      scratch_types=[
          pltpu.VMEM_SHARED((ACC_ROWS, DIM), jnp.float32),
          pltpu.VMEM((SB, BLK), jnp.int32),
          pltpu.VMEM((SB, BLK), jnp.int32),
          pltpu.VMEM((SB, BLK), jnp.int32),
          pltpu.VMEM((SB, BLK), jnp.int32),
          pltpu.VMEM((SB, BLK), jnp.int32),
          pltpu.VMEM((BLK,), jnp.int32),
          pltpu.VMEM((BLK,), jnp.int32),
          pltpu.VMEM((BLK,), jnp.int32),
          pltpu.VMEM((BLK, DIM), jnp.float32),
          pltpu.VMEM((BLK, DIM), jnp.float32),
          pltpu.SemaphoreType.DMA,
          pltpu.SemaphoreType.DMA,
      ],
  )
  def kfn(nfeat_hbm, src_hbm, dst_hbm, ef0_hbm, ef1_hbm, ef2_hbm,
          e0_hbm, e1_hbm, e2_hbm, zf_hbm, pacc_hbm, pdeg_hbm,
          ct_hbm, acc_sh, src_v, dst_v, f0_v, f1_v, f2_v,
          code_v, src1d, dst1d, rows_a, ones_v, sem_a, sem_b):
    c = lax.axis_index("c")
    s = lax.axis_index("s")
    wid = c * NUM_TILES + s

    ones16 = jnp.ones((16,), jnp.float32)
    zeros16 = jnp.zeros((16,), jnp.float32)
    iota16 = lax.iota(jnp.int32, 16)

    base = s * ROWS_PER_TILE
    off = 0
    for n in CHUNKS:
      pltpu.sync_copy(zf_hbm.at[pl.ds(off, n)], rows_a.at[pl.ds(0, n)])
      pltpu.sync_copy(rows_a.at[pl.ds(0, n)], acc_sh.at[pl.ds(base + off, n)])
      off += n

    pltpu.sync_copy(e0_hbm, rows_a.at[pl.ds(16, 5)])
    pltpu.sync_copy(e1_hbm, rows_a.at[pl.ds(24, 5)])
    pltpu.sync_copy(e2_hbm, rows_a.at[pl.ds(32, 5)])

    @pl.loop(0, 8)
    def _(r):
      g = s * 8 + r
      i0 = jnp.minimum(g // 25, 4)
      i1 = (g // 5) % 5
      i2 = g % 5
      for jcol in range(DIM // 16):
        sl = pl.ds(jcol * 16, 16)
        rows_a[r, sl] = (rows_a[16 + i0, sl] + rows_a[24 + i1, sl]
                         + rows_a[32 + i2, sl])

    pltpu.sync_copy(rows_a.at[pl.ds(0, 8)], ct_hbm.at[c, pl.ds(s * 8, 8)])
    plsc.subcore_barrier()

    nblocks = K0 + c * (K1 - K0)
    eb = c * NUM_TILES * K0 + s * nblocks

    @pl.loop(0, nblocks // SB)
    def _(sb):
      sbsl = pl.ds(eb + sb * SB, SB)
      pltpu.sync_copy(src_hbm.at[sbsl], src_v)
      pltpu.sync_copy(dst_hbm.at[sbsl], dst_v)
      pltpu.sync_copy(ef0_hbm.at[sbsl], f0_v)
      pltpu.sync_copy(ef1_hbm.at[sbsl], f1_v)
      pltpu.sync_copy(ef2_hbm.at[sbsl], f2_v)

      @pl.loop(0, SB)
      def _(blk):
        for j in range(BLK // 16):
          sl = pl.ds(j * 16, 16)
          code_v[sl] = f0_v[blk, sl] * 25 + f1_v[blk, sl] * 5 + f2_v[blk, sl]
          src1d[sl] = src_v[blk, sl]
          dst1d[sl] = dst_v[blk, sl]
        cp_a = pltpu.async_copy(nfeat_hbm.at[src1d], rows_a, sem_a)
        cp_b = pltpu.async_copy(ct_hbm.at[c].at[code_v], ones_v, sem_b)
        cp_a.wait()
        cp_b.wait()
        sp_a = pltpu.async_copy(rows_a, acc_sh.at[dst1d], sem_a, add=True)
        sp_b = pltpu.async_copy(ones_v, acc_sh.at[dst1d], sem_b, add=True)
        sp_a.wait()
        sp_b.wait()

    plsc.subcore_barrier()

    off = 0
    for n in CHUNKS:
      sl = pl.ds(base + off, n)
      pltpu.sync_copy(acc_sh.at[sl], rows_a.at[pl.ds(0, n)])
      pltpu.sync_copy(rows_a.at[pl.ds(0, n)], pacc_hbm.at[c, sl])
      off += n

    @pl.loop(0, BLK)
    def _(r):
      for j in range(DIM // 16):
        ones_v[r, pl.ds(j * 16, 16)] = ones16

    off = 0
    for n in CHUNKS:
      pltpu.sync_copy(zf_hbm.at[pl.ds(off, n)], rows_a.at[pl.ds(0, n)])
      pltpu.sync_copy(rows_a.at[pl.ds(0, n)], acc_sh.at[pl.ds(base + off, n)])
      off += n
    plsc.subcore_barrier()

    @pl.loop(0, nblocks // SB)
    def _(sb):
      sbsl = pl.ds(eb + sb * SB, SB)
      pltpu.sync_copy(dst_hbm.at[sbsl], dst_v)

      cps = [None, None]
      bufs = (dst1d, src1d)
      sems = (sem_a, sem_b)
      for b in range(SB):
        p = b % 2
        if cps[p] is not None:
          cps[p].wait()
        for j in range(BLK // 16):
          sl = pl.ds(j * 16, 16)
          bufs[p][sl] = dst_v[b, sl]
        cps[p] = pltpu.async_copy(ones_v, acc_sh.at[bufs[p]], sems[p],
                                  add=True)
      cps[0].wait()
      cps[1].wait()

    plsc.subcore_barrier()

    off = 0
    for n in CHUNKS:
      sl = pl.ds(base + off, n)
      pltpu.sync_copy(acc_sh.at[sl], rows_a.at[pl.ds(0, n)])
      pltpu.sync_copy(rows_a.at[pl.ds(0, n)], pdeg_hbm.at[c, sl])
      off += n

  return kfn(nfeat, src2d, dst2d, ef0, ef1, ef2, e0, e1, e2, zf)


def _tc_finish_body(nfeat_ref, pacc_ref, pdeg0_ref, pdeg1_ref, w_ref, b_ref,
                    out_ref):
  neigh = pacc_ref[0, 0:N_NODES, :] + pacc_ref[1, 0:N_NODES, :]
  deg = pdeg0_ref[0:N_NODES, 0] + pdeg1_ref[0:N_NODES, 0] + 1.0
  h = (nfeat_ref[...] + neigh) / deg[:, None]
  out_ref[...] = (
      lax.dot_general(h, w_ref[...], (((1,), (1,)), ((), ())),
                      precision=lax.Precision.HIGHEST,
                      preferred_element_type=jnp.float32)
      + b_ref[...][None, :]
  )


@jax.jit
def kernel(nfeat, edge_index, efeat, W, b, edge_emb):
  pad = E_PAD - N_EDGES
  src = jnp.pad(edge_index[0].astype(jnp.int32), (0, pad)).reshape(-1, BLK)
  dst = jnp.pad(edge_index[1].astype(jnp.int32), (0, pad),
                constant_values=N_NODES).reshape(-1, BLK)
  ef = efeat.astype(jnp.int32)
  ef0 = jnp.pad(ef[:, 0], (0, pad)).reshape(-1, BLK)
  ef1 = jnp.pad(ef[:, 1], (0, pad)).reshape(-1, BLK)
  ef2 = jnp.pad(ef[:, 2], (0, pad)).reshape(-1, BLK)
  e0 = edge_emb[0]
  e1 = edge_emb[1]
  e2 = edge_emb[2]
  zf = jnp.zeros((ROWS_PER_TILE, DIM), jnp.float32)

  pacc, pdeg, _ct = _sc_accumulate(nfeat, src, dst, ef0, ef1, ef2, e0, e1, e2,
                                   zf)

  out = pl.pallas_call(
      _tc_finish_body,
      out_shape=jax.ShapeDtypeStruct((N_NODES, DIM), jnp.float32),
  )(nfeat, pacc, pdeg[0], pdeg[1], W, b)
  return out
